# baseline (speedup 1.0000x reference)
import functools

import jax
import jax.numpy as jnp
import numpy as np
from jax.experimental import pallas as pl
from jax.experimental.pallas import tpu as pltpu
from jax.experimental.pallas import tpu_sc as plsc

EMBED = 256
HEADS = 8
HEAD_DIM = 32
N_PAIRS = 16
SPATIAL_THETA = 100.0
LEVEL_THETA = 10.0
SIZES = (3, 5, 7, 9)
GRIDS = ((64, 64), (32, 32), (16, 16), (8, 8))
OFFSETS = (0, 4096, 5120, 5376)
TOTAL_TOKENS = 5440
NQ = 512

NKG = 40
DENSE = 320
DOFF = 5120
_ISCALE = 1.0 / (32.0 ** 0.5)


def _lane_iota(rows=1):
    return jax.lax.broadcasted_iota(jnp.int32, (rows, EMBED), 1)


def _freq_lanes():
    pair = (_lane_iota() % HEAD_DIM) // 2
    t = pair.astype(jnp.float32) / float(N_PAIRS)
    fs = jnp.exp(-np.log(SPATIAL_THETA).astype(np.float32) * t)
    fl = jnp.exp(-np.log(LEVEL_THETA).astype(np.float32) * t)
    return fs, fl


def _sign_lanes():
    return jnp.where(_lane_iota() % 2 == 0, -1.0, 1.0).astype(jnp.float32)


def _pswap():
    r = jax.lax.broadcasted_iota(jnp.int32, (EMBED, EMBED), 0)
    c = jax.lax.broadcasted_iota(jnp.int32, (EMBED, EMBED), 1)
    return jnp.where((r ^ 1) == c, 1.0, 0.0).astype(jnp.float32)


def _headsum():
    r = jax.lax.broadcasted_iota(jnp.int32, (EMBED, HEADS), 0)
    c = jax.lax.broadcasted_iota(jnp.int32, (EMBED, HEADS), 1)
    return jnp.where(r // HEAD_DIM == c, 1.0, 0.0).astype(jnp.float32)


def _headexp():
    r = jax.lax.broadcasted_iota(jnp.int32, (HEADS, EMBED), 0)
    c = jax.lax.broadcasted_iota(jnp.int32, (HEADS, EMBED), 1)
    return jnp.where(r == c // HEAD_DIM, 1.0, 0.0).astype(jnp.float32)


def _neigh_offsets(lvl):
    s = SIZES[lvl]
    c = (s - 1) // 2
    j = jax.lax.broadcasted_iota(jnp.int32, (1, s * s), 1)
    return j // s - c, j % s - c


def _pack_bf16_pair(k, v):
    kb = jax.lax.bitcast_convert_type(
        k.astype(jnp.bfloat16).astype(jnp.float32), jnp.int32)
    vb = jax.lax.bitcast_convert_type(
        v.astype(jnp.bfloat16).astype(jnp.float32), jnp.int32)
    return jax.lax.shift_right_logical(kb, 16) | ((vb >> 16) << 16)


def _unpack_bf16_pair(p):
    k = jax.lax.bitcast_convert_type(p << 16, jnp.float32)
    v = jax.lax.bitcast_convert_type((p >> 16) << 16, jnp.float32)
    return k, v


def _lane_swap(x):
    left = jnp.concatenate([x[:, 1:], x[:, :1]], axis=1)
    right = jnp.concatenate([x[:, -1:], x[:, :-1]], axis=1)
    return jnp.where(_lane_iota() % 2 == 0, left, right)


def _token_angles(levels):
    fs, fl = _freq_lanes()
    angs = []
    for lvl in levels:
        h, w = GRIDS[lvl]
        t = jax.lax.broadcasted_iota(jnp.int32, (h * w, 1), 0)
        y = t // w
        x = t - y * w
        pos_sum = (y + x).astype(jnp.float32) * float(2 ** lvl)
        angs.append(pos_sum * fs + float(lvl) * fl)
    return jnp.concatenate(angs, axis=0) if len(angs) > 1 else angs[0]


def _prep_sc_body(fm_ref, wk_ref, wv_ref, pos_ref, tab_ref, idx_ref, pen_ref):
    fm = fm_ref[:DOFF, :].astype(jnp.bfloat16)
    k = jnp.dot(fm, wk_ref[...].astype(jnp.bfloat16),
                preferred_element_type=jnp.float32)
    v = jnp.dot(fm, wv_ref[...].astype(jnp.bfloat16),
                preferred_element_type=jnp.float32)
    ang = _token_angles((0, 1))
    krot = k * jnp.cos(ang) + _sign_lanes() * _lane_swap(k) * jnp.sin(ang)
    tab_ref[...] = _pack_bf16_pair(krot, v)

    py = pos_ref[:, 0:1]
    px = pos_ref[:, 1:2]
    idx_parts = []
    pen_parts = []
    for lvl in (0, 1):
        h, w = GRIDS[lvl]
        scale = float(2.0 ** (-lvl))
        by = jnp.floor(py * scale).astype(jnp.int32)
        bx = jnp.floor(px * scale).astype(jnp.int32)
        oy, ox = _neigh_offsets(lvl)
        iy = by + oy
        ix = bx + ox
        oob = (iy < 0) | (iy >= h) | (ix < 0) | (ix >= w)
        iyc = jnp.clip(iy, 0, h - 1)
        ixc = jnp.clip(ix, 0, w - 1)
        idx_parts.append(OFFSETS[lvl] + iyc * w + ixc)
        pen_parts.append(jnp.where(oob, -1e9, 0.0).astype(jnp.float32))
    npad = NKG - 34
    idx_parts.append(jnp.zeros((NQ, npad), jnp.int32))
    pen_parts.append(jnp.full((NQ, npad), -1e9, jnp.float32))
    idx_ref[...] = jnp.concatenate(idx_parts, axis=1)
    pen_ref[...] = jnp.concatenate(pen_parts, axis=1)


def _prep_sc(feature_maps, Wk, Wv, pos):
    return pl.pallas_call(
        _prep_sc_body,
        out_shape=[
            jax.ShapeDtypeStruct((DOFF, EMBED), jnp.int32),
            jax.ShapeDtypeStruct((NQ, NKG), jnp.int32),
            jax.ShapeDtypeStruct((NQ, NKG), jnp.float32),
        ],
    )(feature_maps, Wk, Wv, pos)


def _overlap_body(fm_ref, wk_ref, wv_ref, q_ref, pos_ref, g_ref, b_ref,
                  wq_ref, qrot_ref, m2e_ref, s2e_ref, o2_ref):
    fm_d = fm_ref[DOFF:, :]
    k_d = jnp.dot(fm_d, wk_ref[...], preferred_element_type=jnp.float32)
    v_d = jnp.dot(fm_d, wv_ref[...], preferred_element_type=jnp.float32)
    ang = _token_angles((2, 3))
    krot_d = (k_d * jnp.cos(ang)
              + _sign_lanes() * _lane_swap(k_d) * jnp.sin(ang))
    kdt = krot_d.T

    x = q_ref[...]
    mu = jnp.mean(x, axis=-1, keepdims=True)
    var = jnp.mean((x - mu) * (x - mu), axis=-1, keepdims=True)
    xn = (x - mu) * jax.lax.rsqrt(var + 1e-5) * g_ref[...] + b_ref[...]
    q = jnp.dot(xn, wq_ref[...], preferred_element_type=jnp.float32)
    py = pos_ref[:, 0:1]
    px = pos_ref[:, 1:2]
    fs, _ = _freq_lanes()
    ang_q = (py + px) * fs
    q = (q * jnp.cos(ang_q) + _sign_lanes() * _lane_swap(q) * jnp.sin(ang_q))
    qrot_ref[...] = q

    hexp = _headexp()
    r_i = jax.lax.broadcasted_iota(jnp.int32, (NQ * HEADS, EMBED), 0)
    e_i = jax.lax.broadcasted_iota(jnp.int32, (NQ * HEADS, EMBED), 1)
    hm = jnp.where(r_i % HEADS == e_i // HEAD_DIM, 1.0, 0.0)
    q_rep = jnp.broadcast_to(q[:, None, :], (NQ, HEADS, EMBED)).reshape(
        NQ * HEADS, EMBED)
    ld = jnp.dot(q_rep * hm, kdt,
                 preferred_element_type=jnp.float32)

    by2 = jnp.floor(py * 0.25).astype(jnp.int32)
    bx2 = jnp.floor(px * 0.25).astype(jnp.int32)
    by3 = jnp.floor(py * 0.125).astype(jnp.int32)
    bx3 = jnp.floor(px * 0.125).astype(jnp.int32)

    def rep(a):
        return jnp.broadcast_to(a[:, None, :], (NQ, HEADS, 1)).reshape(
            NQ * HEADS, 1)

    t = jax.lax.broadcasted_iota(jnp.int32, (1, DENSE), 1)
    y2 = t // 16
    x2 = t % 16
    u = t - 256
    y3 = u // 8
    x3 = u % 8
    in2 = ((t < 256) & (jnp.abs(y2 - rep(by2)) <= 3)
           & (jnp.abs(x2 - rep(bx2)) <= 3))
    in3 = ((t >= 256) & (jnp.abs(y3 - rep(by3)) <= 4)
           & (jnp.abs(x3 - rep(bx3)) <= 4))
    ld = ld * _ISCALE + jnp.where(in2 | in3, 0.0, -1e9)

    m2 = jnp.max(ld, axis=1, keepdims=True)
    e2 = jnp.exp(ld - m2)
    s2 = jnp.sum(e2, axis=1, keepdims=True)
    o2full = jnp.dot(e2, v_d,
                     preferred_element_type=jnp.float32)
    h_i = jax.lax.broadcasted_iota(jnp.int32, (NQ, HEADS, EMBED), 1)
    eh_i = jax.lax.broadcasted_iota(jnp.int32, (NQ, HEADS, EMBED), 2)
    hm3 = jnp.where(h_i == eh_i // HEAD_DIM, 1.0, 0.0)
    o2_ref[...] = jnp.sum(o2full.reshape(NQ, HEADS, EMBED) * hm3, axis=1)
    m2e_ref[...] = jnp.dot(m2.reshape(NQ, HEADS), hexp,
                           preferred_element_type=jnp.float32)
    s2e_ref[...] = jnp.dot(s2.reshape(NQ, HEADS), hexp,
                           preferred_element_type=jnp.float32)


def _overlap(feature_maps, Wk, Wv, query, pos, gamma, beta, Wq):
    return pl.pallas_call(
        _overlap_body,
        out_shape=[
            jax.ShapeDtypeStruct((NQ, EMBED), jnp.float32),
            jax.ShapeDtypeStruct((NQ, EMBED), jnp.float32),
            jax.ShapeDtypeStruct((NQ, EMBED), jnp.float32),
            jax.ShapeDtypeStruct((NQ, EMBED), jnp.float32),
        ],
    )(feature_maps, Wk, Wv, query, pos,
      gamma.reshape(1, EMBED), beta.reshape(1, EMBED), Wq)


_GATHER_WINDOW = 128


def _sc_gather(tab, idx_flat):
    n = NQ * NKG
    mesh = plsc.VectorSubcoreMesh(core_axis_name="core",
                                  subcore_axis_name="subcore")

    @functools.partial(
        pl.kernel,
        out_type=jax.ShapeDtypeStruct((n, EMBED), jnp.int32),
        mesh=mesh,
    )
    def gather_kernel(t_hbm, i_hbm, o_hbm):
        def body(i_vmem, o_vmem):
            pltpu.sync_copy(t_hbm.at[i_vmem.at[0]], o_vmem)

        pltpu.emit_pipeline(
            body,
            grid=(n // _GATHER_WINDOW,),
            in_specs=[pl.BlockSpec((1, _GATHER_WINDOW), lambda i: (0, i))],
            out_specs=[
                pl.BlockSpec((_GATHER_WINDOW, EMBED), lambda i: (i, 0)),
            ],
            core_axis_name=("core", "subcore"),
            dimension_semantics=(pltpu.PARALLEL,),
        )(i_hbm, o_hbm)

    return gather_kernel(tab, idx_flat)


_BQ = 32


def _final_body(q_ref, kv_ref, pen_ref, m2e_ref, s2e_ref, o2_ref,
                res_ref, wo_ref, o_ref):
    q = q_ref[...]
    hexp = _headexp()
    kg, vg = _unpack_bf16_pair(kv_ref[...])
    p = kg * q[:, None, :]
    l1 = jnp.dot(p.reshape(_BQ * NKG, EMBED), _headsum(),
                 preferred_element_type=jnp.float32).reshape(_BQ, NKG, HEADS)
    l1 = l1 * _ISCALE + pen_ref[...][:, :, None]
    m1 = jnp.max(l1, axis=1, keepdims=True)
    e1 = jnp.exp(l1 - m1)
    s1 = jnp.sum(e1, axis=1, keepdims=True)
    ew = jnp.dot(e1.reshape(_BQ * NKG, HEADS), hexp,
                 preferred_element_type=jnp.float32).reshape(_BQ, NKG, EMBED)
    o1 = jnp.sum(ew * vg, axis=1)
    m1e = jnp.dot(m1.reshape(_BQ, HEADS), hexp,
                  preferred_element_type=jnp.float32)
    s1e = jnp.dot(s1.reshape(_BQ, HEADS), hexp,
                  preferred_element_type=jnp.float32)

    m2e = m2e_ref[...]
    s2e = s2e_ref[...]
    o2 = o2_ref[...]
    mm = jnp.maximum(m1e, m2e)
    a1 = jnp.exp(m1e - mm)
    a2 = jnp.exp(m2e - mm)
    out = (o1 * a1 + o2 * a2) / (s1e * a1 + s2e * a2)
    o_ref[...] = res_ref[...] + jnp.dot(out, wo_ref[...],
                                        preferred_element_type=jnp.float32)


def _final(qrot, gkv, pen, m2e, s2e, o2, residual, Wo):
    return pl.pallas_call(
        _final_body,
        grid=(NQ // _BQ,),
        in_specs=[
            pl.BlockSpec((_BQ, EMBED), lambda i: (i, 0)),
            pl.BlockSpec((_BQ, NKG, EMBED), lambda i: (i, 0, 0)),
            pl.BlockSpec((_BQ, NKG), lambda i: (i, 0)),
            pl.BlockSpec((_BQ, EMBED), lambda i: (i, 0)),
            pl.BlockSpec((_BQ, EMBED), lambda i: (i, 0)),
            pl.BlockSpec((_BQ, EMBED), lambda i: (i, 0)),
            pl.BlockSpec((_BQ, EMBED), lambda i: (i, 0)),
            pl.BlockSpec((EMBED, EMBED), lambda i: (0, 0)),
        ],
        out_specs=pl.BlockSpec((_BQ, EMBED), lambda i: (i, 0)),
        out_shape=jax.ShapeDtypeStruct((NQ, EMBED), jnp.float32),
    )(qrot, gkv, pen, m2e, s2e, o2, residual, Wo)


@jax.jit
def kernel(query, query_spatial_positions, feature_maps, level_spatial_shapes,
           ln_gamma, ln_beta, Wq, Wk, Wv, Wo):
    del level_spatial_shapes
    tab, idx, pen = _prep_sc(feature_maps, Wk, Wv, query_spatial_positions)
    gkv = _sc_gather(tab, idx.reshape(1, NQ * NKG))
    qrot, m2e, s2e, o2 = _overlap(feature_maps, Wk, Wv, query,
                                  query_spatial_positions, ln_gamma,
                                  ln_beta, Wq)
    gkv = gkv.reshape(NQ, NKG, EMBED)
    return _final(qrot, gkv, pen, m2e, s2e, o2, query, Wo)


# speedup vs baseline: 1.0355x; 1.0355x over previous
import functools

import jax
import jax.numpy as jnp
import numpy as np
from jax.experimental import pallas as pl
from jax.experimental.pallas import tpu as pltpu
from jax.experimental.pallas import tpu_sc as plsc

EMBED = 256
HEADS = 8
HEAD_DIM = 32
N_PAIRS = 16
SPATIAL_THETA = 100.0
LEVEL_THETA = 10.0
SIZES = (3, 5, 7, 9)
GRIDS = ((64, 64), (32, 32), (16, 16), (8, 8))
OFFSETS = (0, 4096, 5120, 5376)
TOTAL_TOKENS = 5440
NQ = 512
NKG = 40
DENSE = 320
DOFF = 5120


def _lane_iota(rows=1):
    return jax.lax.broadcasted_iota(jnp.int32, (rows, EMBED), 1)


def _freq_lanes():
    pair = (_lane_iota() % HEAD_DIM) // 2
    t = pair.astype(jnp.float32) / float(N_PAIRS)
    fs = jnp.exp(-np.log(SPATIAL_THETA).astype(np.float32) * t)
    fl = jnp.exp(-np.log(LEVEL_THETA).astype(np.float32) * t)
    return fs, fl


def _sign_lanes():
    return jnp.where(_lane_iota() % 2 == 0, -1.0, 1.0).astype(jnp.float32)


def _pswap():
    r = jax.lax.broadcasted_iota(jnp.int32, (EMBED, EMBED), 0)
    c = jax.lax.broadcasted_iota(jnp.int32, (EMBED, EMBED), 1)
    return jnp.where((r ^ 1) == c, 1.0, 0.0).astype(jnp.float32)


def _headsum():
    r = jax.lax.broadcasted_iota(jnp.int32, (EMBED, HEADS), 0)
    c = jax.lax.broadcasted_iota(jnp.int32, (EMBED, HEADS), 1)
    return jnp.where(r // HEAD_DIM == c, 1.0, 0.0).astype(jnp.float32)


def _headexp():
    r = jax.lax.broadcasted_iota(jnp.int32, (HEADS, EMBED), 0)
    c = jax.lax.broadcasted_iota(jnp.int32, (HEADS, EMBED), 1)
    return jnp.where(r == c // HEAD_DIM, 1.0, 0.0).astype(jnp.float32)


def _neigh_offsets(lvl):
    s = SIZES[lvl]
    c = (s - 1) // 2
    j = jax.lax.broadcasted_iota(jnp.int32, (1, s * s), 1)
    return j // s - c, j % s - c


def _pack_bf16_pair(k, v):
    kb = jax.lax.bitcast_convert_type(
        k.astype(jnp.bfloat16).astype(jnp.float32), jnp.int32)
    vb = jax.lax.bitcast_convert_type(
        v.astype(jnp.bfloat16).astype(jnp.float32), jnp.int32)
    return jax.lax.shift_right_logical(kb, 16) | ((vb >> 16) << 16)


def _unpack_bf16_pair(p):
    k = jax.lax.bitcast_convert_type(p << 16, jnp.float32)
    v = jax.lax.bitcast_convert_type((p >> 16) << 16, jnp.float32)
    return k, v


def _lane_swap(x):
    left = jnp.concatenate([x[:, 1:], x[:, :1]], axis=1)
    right = jnp.concatenate([x[:, -1:], x[:, :-1]], axis=1)
    return jnp.where(_lane_iota() % 2 == 0, left, right)


def _token_angles(levels):
    fs, fl = _freq_lanes()
    angs = []
    for lvl in levels:
        h, w = GRIDS[lvl]
        t = jax.lax.broadcasted_iota(jnp.int32, (h * w, 1), 0)
        y = t // w
        x = t - y * w
        pos_sum = (y + x).astype(jnp.float32) * float(2 ** lvl)
        angs.append(pos_sum * fs + float(lvl) * fl)
    return jnp.concatenate(angs, axis=0) if len(angs) > 1 else angs[0]


def _prep_sc_body(fm_ref, wk_ref, wv_ref, pos_ref, tab_ref, idx_ref, pen_ref):
    fm = fm_ref[:DOFF, :].astype(jnp.bfloat16)
    k = jnp.dot(fm, wk_ref[...].astype(jnp.bfloat16),
                preferred_element_type=jnp.float32)
    v = jnp.dot(fm, wv_ref[...].astype(jnp.bfloat16),
                preferred_element_type=jnp.float32)
    ang = _token_angles((0, 1))
    krot = k * jnp.cos(ang) + _sign_lanes() * _lane_swap(k) * jnp.sin(ang)
    tab_ref[...] = _pack_bf16_pair(krot, v)

    py = pos_ref[:, 0:1]
    px = pos_ref[:, 1:2]
    idx_parts = []
    pen_parts = []
    for lvl in (0, 1):
        h, w = GRIDS[lvl]
        scale = float(2.0 ** (-lvl))
        by = jnp.floor(py * scale).astype(jnp.int32)
        bx = jnp.floor(px * scale).astype(jnp.int32)
        oy, ox = _neigh_offsets(lvl)
        iy = by + oy
        ix = bx + ox
        oob = (iy < 0) | (iy >= h) | (ix < 0) | (ix >= w)
        iyc = jnp.clip(iy, 0, h - 1)
        ixc = jnp.clip(ix, 0, w - 1)
        idx_parts.append(OFFSETS[lvl] + iyc * w + ixc)
        pen_parts.append(jnp.where(oob, -1e9, 0.0).astype(jnp.float32))
    npad = NKG - 34
    idx_parts.append(jnp.zeros((NQ, npad), jnp.int32))
    pen_parts.append(jnp.full((NQ, npad), -1e9, jnp.float32))
    idx_ref[...] = jnp.concatenate(idx_parts, axis=1)
    pen_ref[...] = jnp.concatenate(pen_parts, axis=1)


def _prep_sc(feature_maps, Wk, Wv, pos):
    return pl.pallas_call(
        _prep_sc_body,
        out_shape=[
            jax.ShapeDtypeStruct((DOFF, EMBED), jnp.int32),
            jax.ShapeDtypeStruct((NQ, NKG), jnp.int32),
            jax.ShapeDtypeStruct((NQ, NKG), jnp.float32),
        ],
    )(feature_maps, Wk, Wv, pos)


def _prep_rest_body(fm_ref, wk_ref, wv_ref, q_ref, pos_ref, g_ref, b_ref,
                    wq_ref, qrot_ref, kdt_ref, vd_ref):
    fm_d = fm_ref[DOFF:, :]
    k_d = jnp.dot(fm_d, wk_ref[...], preferred_element_type=jnp.float32)
    v_d = jnp.dot(fm_d, wv_ref[...], preferred_element_type=jnp.float32)
    ang = _token_angles((2, 3))
    krot_d = (k_d * jnp.cos(ang)
              + _sign_lanes() * _lane_swap(k_d) * jnp.sin(ang))
    kdt_ref[...] = krot_d.T
    vd_ref[...] = v_d

    x = q_ref[...]
    mu = jnp.mean(x, axis=-1, keepdims=True)
    var = jnp.mean((x - mu) * (x - mu), axis=-1, keepdims=True)
    xn = (x - mu) * jax.lax.rsqrt(var + 1e-5) * g_ref[...] + b_ref[...]
    q = jnp.dot(xn, wq_ref[...], preferred_element_type=jnp.float32)
    py = pos_ref[:, 0:1]
    px = pos_ref[:, 1:2]
    fs, _ = _freq_lanes()
    ang_q = (py + px) * fs
    qrot_ref[...] = (q * jnp.cos(ang_q)
                     + _sign_lanes() * _lane_swap(q) * jnp.sin(ang_q))


def _prep_rest(feature_maps, Wk, Wv, query, pos, gamma, beta, Wq):
    return pl.pallas_call(
        _prep_rest_body,
        out_shape=[
            jax.ShapeDtypeStruct((NQ, EMBED), jnp.float32),
            jax.ShapeDtypeStruct((EMBED, DENSE), jnp.float32),
            jax.ShapeDtypeStruct((DENSE, EMBED), jnp.float32),
        ],
    )(feature_maps, Wk, Wv, query, pos,
      gamma.reshape(1, EMBED), beta.reshape(1, EMBED), Wq)


_GATHER_WINDOW = 128


def _sc_gather(tab, idx_flat):
    n = NQ * NKG
    mesh = plsc.VectorSubcoreMesh(core_axis_name="core",
                                  subcore_axis_name="subcore")

    @functools.partial(
        pl.kernel,
        out_type=jax.ShapeDtypeStruct((n, EMBED), jnp.int32),
        mesh=mesh,
    )
    def gather_kernel(t_hbm, i_hbm, o_hbm):
        def body(i_vmem, o_vmem):
            pltpu.sync_copy(t_hbm.at[i_vmem.at[0]], o_vmem)

        pltpu.emit_pipeline(
            body,
            grid=(n // _GATHER_WINDOW,),
            in_specs=[pl.BlockSpec((1, _GATHER_WINDOW), lambda i: (0, i))],
            out_specs=[
                pl.BlockSpec((_GATHER_WINDOW, EMBED), lambda i: (i, 0)),
            ],
            core_axis_name=("core", "subcore"),
            dimension_semantics=(pltpu.PARALLEL,),
        )(i_hbm, o_hbm)

    return gather_kernel(tab, idx_flat)


_BQ = 64


_ISCALE = 1.0 / np.sqrt(HEAD_DIM)
_BD = 32


def _dense_body(q_ref, pos_ref, kdt_ref, vd_ref, m2e_ref, s2e_ref, o2_ref):
    q = q_ref[...]
    hexp = _headexp()
    r_i = jax.lax.broadcasted_iota(jnp.int32, (_BD * HEADS, EMBED), 0)
    e_i = jax.lax.broadcasted_iota(jnp.int32, (_BD * HEADS, EMBED), 1)
    hm = jnp.where(r_i % HEADS == e_i // HEAD_DIM, 1.0, 0.0)
    q_rep = jnp.broadcast_to(q[:, None, :], (_BD, HEADS, EMBED)).reshape(
        _BD * HEADS, EMBED)
    ld = jnp.dot(q_rep * hm, kdt_ref[...],
                 preferred_element_type=jnp.float32)

    py = pos_ref[:, 0:1]
    px = pos_ref[:, 1:2]
    by2 = jnp.floor(py * 0.25).astype(jnp.int32)
    bx2 = jnp.floor(px * 0.25).astype(jnp.int32)
    by3 = jnp.floor(py * 0.125).astype(jnp.int32)
    bx3 = jnp.floor(px * 0.125).astype(jnp.int32)

    def rep(a):
        return jnp.broadcast_to(a[:, None, :], (_BD, HEADS, 1)).reshape(
            _BD * HEADS, 1)

    t = jax.lax.broadcasted_iota(jnp.int32, (1, DENSE), 1)
    y2 = t // 16
    x2 = t % 16
    u = t - 256
    y3 = u // 8
    x3 = u % 8
    in2 = ((t < 256) & (jnp.abs(y2 - rep(by2)) <= 3)
           & (jnp.abs(x2 - rep(bx2)) <= 3))
    in3 = ((t >= 256) & (jnp.abs(y3 - rep(by3)) <= 4)
           & (jnp.abs(x3 - rep(bx3)) <= 4))
    ld = ld * _ISCALE + jnp.where(in2 | in3, 0.0, -1e9)

    m2 = jnp.max(ld, axis=1, keepdims=True)
    e2 = jnp.exp(ld - m2)
    s2 = jnp.sum(e2, axis=1, keepdims=True)
    o2full = jnp.dot(e2, vd_ref[...],
                     preferred_element_type=jnp.float32)
    h_i = jax.lax.broadcasted_iota(jnp.int32, (_BD, HEADS, EMBED), 1)
    eh_i = jax.lax.broadcasted_iota(jnp.int32, (_BD, HEADS, EMBED), 2)
    hm3 = jnp.where(h_i == eh_i // HEAD_DIM, 1.0, 0.0)
    o2_ref[...] = jnp.sum(o2full.reshape(_BD, HEADS, EMBED) * hm3, axis=1)
    m2e_ref[...] = jnp.dot(m2.reshape(_BD, HEADS), hexp,
                           preferred_element_type=jnp.float32)
    s2e_ref[...] = jnp.dot(s2.reshape(_BD, HEADS), hexp,
                           preferred_element_type=jnp.float32)


def _dense_attention(qrot, pos, kdt, vd):
    return pl.pallas_call(
        _dense_body,
        grid=(NQ // _BD,),
        in_specs=[
            pl.BlockSpec((_BD, EMBED), lambda i: (i, 0)),
            pl.BlockSpec((_BD, 2), lambda i: (i, 0)),
            pl.BlockSpec((EMBED, DENSE), lambda i: (0, 0)),
            pl.BlockSpec((DENSE, EMBED), lambda i: (0, 0)),
        ],
        out_specs=[
            pl.BlockSpec((_BD, EMBED), lambda i: (i, 0)),
            pl.BlockSpec((_BD, EMBED), lambda i: (i, 0)),
            pl.BlockSpec((_BD, EMBED), lambda i: (i, 0)),
        ],
        out_shape=[
            jax.ShapeDtypeStruct((NQ, EMBED), jnp.float32),
            jax.ShapeDtypeStruct((NQ, EMBED), jnp.float32),
            jax.ShapeDtypeStruct((NQ, EMBED), jnp.float32),
        ],
    )(qrot, pos, kdt, vd)


def _final_body(q_ref, kv_ref, pen_ref, m2e_ref, s2e_ref, o2_ref,
                res_ref, wo_ref, o_ref):
    q = q_ref[...]
    hexp = _headexp()
    kg, vg = _unpack_bf16_pair(kv_ref[...])
    p = kg.astype(jnp.bfloat16) * q.astype(jnp.bfloat16)[:, None, :]
    l1 = jnp.dot(p.reshape(_BQ * NKG, EMBED),
                 _headsum().astype(jnp.bfloat16),
                 preferred_element_type=jnp.float32).reshape(_BQ, NKG, HEADS)
    l1 = l1 * _ISCALE + pen_ref[...][:, :, None]
    m1 = jnp.max(l1, axis=1, keepdims=True)
    e1 = jnp.exp(l1 - m1)
    s1 = jnp.sum(e1, axis=1, keepdims=True)
    ew = jnp.dot(e1.reshape(_BQ * NKG, HEADS), hexp,
                 preferred_element_type=jnp.float32).reshape(_BQ, NKG, EMBED)
    o1 = jnp.sum(ew * vg, axis=1)
    m1e = jnp.dot(m1.reshape(_BQ, HEADS), hexp,
                  preferred_element_type=jnp.float32)
    s1e = jnp.dot(s1.reshape(_BQ, HEADS), hexp,
                  preferred_element_type=jnp.float32)

    m2e = m2e_ref[...]
    s2e = s2e_ref[...]
    o2 = o2_ref[...]
    mm = jnp.maximum(m1e, m2e)
    a1 = jnp.exp(m1e - mm)
    a2 = jnp.exp(m2e - mm)
    out = (o1 * a1 + o2 * a2) / (s1e * a1 + s2e * a2)
    o_ref[...] = res_ref[...] + jnp.dot(out, wo_ref[...],
                                        preferred_element_type=jnp.float32)


def _final(qrot, gkv, pen, m2e, s2e, o2, residual, Wo):
    return pl.pallas_call(
        _final_body,
        grid=(NQ // _BQ,),
        in_specs=[
            pl.BlockSpec((_BQ, EMBED), lambda i: (i, 0)),
            pl.BlockSpec((_BQ, NKG, EMBED), lambda i: (i, 0, 0)),
            pl.BlockSpec((_BQ, NKG), lambda i: (i, 0)),
            pl.BlockSpec((_BQ, EMBED), lambda i: (i, 0)),
            pl.BlockSpec((_BQ, EMBED), lambda i: (i, 0)),
            pl.BlockSpec((_BQ, EMBED), lambda i: (i, 0)),
            pl.BlockSpec((_BQ, EMBED), lambda i: (i, 0)),
            pl.BlockSpec((EMBED, EMBED), lambda i: (0, 0)),
        ],
        out_specs=pl.BlockSpec((_BQ, EMBED), lambda i: (i, 0)),
        out_shape=jax.ShapeDtypeStruct((NQ, EMBED), jnp.float32),
    )(qrot, gkv, pen, m2e, s2e, o2, residual, Wo)


@jax.jit
def kernel(query, query_spatial_positions, feature_maps, level_spatial_shapes,
           ln_gamma, ln_beta, Wq, Wk, Wv, Wo):
    del level_spatial_shapes
    tab, idx, pen = _prep_sc(feature_maps, Wk, Wv, query_spatial_positions)
    gkv = _sc_gather(tab, idx.reshape(1, NQ * NKG))
    qrot, kdt, vd = _prep_rest(feature_maps, Wk, Wv, query,
                               query_spatial_positions, ln_gamma, ln_beta, Wq)
    m2e, s2e, o2 = _dense_attention(qrot, query_spatial_positions, kdt, vd)
    gkv = gkv.reshape(NQ, NKG, EMBED)
    return _final(qrot, gkv, pen, m2e, s2e, o2, query, Wo)
